# Initial kernel scaffold; baseline (speedup 1.0000x reference)
import jax
import jax.numpy as jnp
from jax.experimental import pallas as pl


def kernel(x, edge_index, W1, b1, W2, b2, Wm1, bm1, Wm2, bm2):
    raise NotImplementedError("write your pallas kernel here")


# speedup vs baseline: 15.2475x; 15.2475x over previous
import functools

import jax
import jax.numpy as jnp
from jax import lax
from jax.experimental import pallas as pl
from jax.experimental.pallas import tpu as pltpu
from jax.experimental.pallas import tpu_sc as plsc

NC = 2
NS = 16
NW = NC * NS
F32 = jnp.float32

N = 10000
E = 320000
D_IN = 128
H = 256
HF = H // NC
NP = 10240
RPW = NP // NS
K = 80

_MESH = plsc.VectorSubcoreMesh(
    core_axis_name="c", subcore_axis_name="s", num_cores=NC, num_subcores=NS)


def _worker(c, s):
    return s * NC + c


HB = 25


@functools.partial(
    pl.kernel,
    out_type=jax.ShapeDtypeStruct((NC * NP,), F32),
    mesh=_MESH,
    scratch_types=[
        pltpu.VMEM_SHARED((NP,), F32),
        pltpu.VMEM((HB * K,), jnp.int32),
        pltpu.VMEM((2, K), jnp.int32),
        pltpu.VMEM((K,), F32),
        pltpu.SemaphoreType.DMA,
        pltpu.SemaphoreType.DMA,
    ],
)
def _hist_k(dst_hbm, z1_hbm, out_hbm, hist_sp, idx_blk, idx2, ones_v,
            sem0, sem1):
    c = lax.axis_index("c")
    s = lax.axis_index("s")
    w = _worker(c, s)
    ew = E // NW
    pltpu.sync_copy(z1_hbm, hist_sp.at[pl.ds(s * RPW, RPW)])
    for j in range(K // 16):
        ones_v[pl.ds(j * 16, 16)] = jnp.ones((16,), F32)
    plsc.subcore_barrier()
    sems = (sem0, sem1)

    def issue(kk, b):
        ib = idx2.at[b]
        for j in range(K // 16):
            ib[pl.ds(j * 16, 16)] = idx_blk[pl.ds(kk * K + j * 16, 16)]
        pltpu.async_copy(ones_v, hist_sp.at[ib], sems[b], add=True)

    def drain(b):
        pltpu.make_async_copy(ones_v, hist_sp.at[idx2.at[b]],
                              sems[b]).wait()

    def block(nb, carry):
        pltpu.sync_copy(dst_hbm.at[pl.ds(w * ew + nb * (HB * K), HB * K)],
                        idx_blk)
        issue(0, 0)
        issue(1, 1)

        def body(k2, carry2):
            for b in range(2):
                kk = k2 * 2 + b
                drain(b)

                @pl.when(kk + 2 < HB)
                def _():
                    issue(kk + 2, b)
            return carry2

        lax.fori_loop(0, (HB - 1) // 2, body, 0)
        drain(0)
        return carry

    lax.fori_loop(0, ew // K // HB, block, 0)
    plsc.subcore_barrier()
    pltpu.sync_copy(hist_sp.at[pl.ds(s * RPW, RPW)],
                    out_hbm.at[pl.ds(c * NP + s * RPW, RPW)])


EW = E // NS
CH = EW // K
IB = 25
NB = CH // IB


@functools.partial(
    pl.kernel,
    out_type=jax.ShapeDtypeStruct((NC * NP, HF), F32),
    mesh=_MESH,
    scratch_types=[
        pltpu.VMEM_SHARED((NP, HF), F32),
        pltpu.VMEM((IB * K,), jnp.int32),
        pltpu.VMEM((IB * K,), jnp.int32),
        pltpu.VMEM((4, K), jnp.int32),
        pltpu.VMEM((2, K), jnp.int32),
        pltpu.VMEM((4, K, HF), F32),
        pltpu.SemaphoreType.DMA,
        pltpu.SemaphoreType.DMA,
        pltpu.SemaphoreType.DMA,
        pltpu.SemaphoreType.DMA,
        pltpu.SemaphoreType.DMA,
        pltpu.SemaphoreType.DMA,
    ],
)
def _scatter_k(table_hbm, src_hbm, dst_hbm, z_hbm, out_hbm,
               acc_sp, sidx_blk, didx_blk, gidx, didx2, rows,
               semg0, semg1, semg2, semg3, semsc0, semsc1):
    c = lax.axis_index("c")
    s = lax.axis_index("s")
    pltpu.sync_copy(z_hbm, acc_sp.at[pl.ds(s * RPW, RPW)])
    plsc.subcore_barrier()
    roff = c * NP
    semg = (semg0, semg1, semg2, semg3)
    semsc = (semsc0, semsc1)

    def issue_g(kk, g):
        gb = gidx.at[g]
        for j in range(K // 16):
            gb[pl.ds(j * 16, 16)] = \
                sidx_blk[pl.ds(kk * K + j * 16, 16)] + roff
        pltpu.async_copy(table_hbm.at[gb], rows.at[g], semg[g])

    def drain_g(g):
        pltpu.make_async_copy(table_hbm.at[gidx.at[g]],
                              rows.at[g], semg[g]).wait()

    def issue_sc(kk, g, b):
        db = didx2.at[b]
        for j in range(K // 16):
            db[pl.ds(j * 16, 16)] = \
                didx_blk[pl.ds(kk * K + j * 16, 16)]
        pltpu.async_copy(rows.at[g], acc_sp.at[db], semsc[b], add=True)

    def drain_sc(g, b):
        pltpu.make_async_copy(rows.at[g], acc_sp.at[didx2.at[b]],
                              semsc[b]).wait()

    def step(kk, g):
        b = g % 2
        drain_g(g)

        @pl.when(kk >= 2)
        def _():
            drain_sc((g + 2) % 4, b)

        @pl.when(kk + 2 < IB)
        def _():
            issue_g(kk + 2, (g + 2) % 4)

        issue_sc(kk, g, b)

    def block(nb, carry):
        bb = s * EW + nb * (IB * K)
        pltpu.sync_copy(src_hbm.at[pl.ds(bb, IB * K)], sidx_blk)
        pltpu.sync_copy(dst_hbm.at[pl.ds(bb, IB * K)], didx_blk)
        issue_g(0, 0)
        issue_g(1, 1)

        def quad(q, carry2):
            for b4 in range(4):
                step(q * 4 + b4, b4)
            return carry2

        lax.fori_loop(0, IB // 4, quad, 0)
        step(IB - 1, (IB - 1) % 4)
        drain_sc(0, 0)
        drain_sc(3, 1)
        return carry

    lax.fori_loop(0, NB, block, 0)
    plsc.subcore_barrier()
    pltpu.sync_copy(acc_sp.at[pl.ds(s * RPW, RPW)],
                    out_hbm.at[pl.ds(roff + s * RPW, RPW)])


EW2 = E // NW
CH2 = EW2 // K


@functools.partial(
    pl.kernel,
    out_type=[jax.ShapeDtypeStruct((E * 16,), F32),
              jax.ShapeDtypeStruct((E * 16,), F32)],
    mesh=_MESH,
    scratch_types=[
        pltpu.VMEM((EW2,), jnp.int32),
        pltpu.VMEM((EW2,), jnp.int32),
        pltpu.VMEM((2, K, H), F32),
        pltpu.VMEM((2, K, H), F32),
        pltpu.VMEM((2, K * 16), F32),
        pltpu.VMEM((2, K * 16), F32),
        pltpu.VMEM((NC, H), F32),
        pltpu.SemaphoreType.DMA,
        pltpu.SemaphoreType.DMA,
        pltpu.SemaphoreType.DMA,
        pltpu.SemaphoreType.DMA,
    ],
)
def _edge_mlp_k(p_hbm, src4_hbm, dst4_hbm, wm2t_hbm, out0_hbm, out1_hbm,
                sidx_all, didx_all, ps, pd, o0, o1, wv,
                semg0, semg1, semo0, semo1):
    c = lax.axis_index("c")
    s = lax.axis_index("s")
    w = _worker(c, s)
    pltpu.sync_copy(wm2t_hbm, wv)
    pltpu.sync_copy(src4_hbm.at[w], sidx_all)
    pltpu.sync_copy(dst4_hbm.at[w], didx_all)
    w0 = [wv[0, pl.ds(j * 16, 16)] for j in range(H // 16)]
    w1 = [wv[1, pl.ds(j * 16, 16)] for j in range(H // 16)]
    semg = (semg0, semg1)
    semo = (semo0, semo1)

    def issue(k, b):
        pltpu.async_copy(p_hbm.at[sidx_all.at[pl.ds(k * K, K)]],
                         ps.at[b], semg[b])
        pltpu.async_copy(p_hbm.at[didx_all.at[pl.ds(k * K, K)]],
                         pd.at[b], semg[b])

    def drain_g(b):
        pltpu.make_async_copy(p_hbm.at[sidx_all.at[pl.ds(0, K)]],
                              ps.at[b], semg[b]).wait()
        pltpu.make_async_copy(p_hbm.at[didx_all.at[pl.ds(0, K)]],
                              pd.at[b], semg[b]).wait()

    def issue_out(k, b):
        base = (w * EW2 + k * K) * 16
        pltpu.async_copy(o0.at[b], out0_hbm.at[pl.ds(base, K * 16)], semo[b])
        pltpu.async_copy(o1.at[b], out1_hbm.at[pl.ds(base, K * 16)], semo[b])

    def drain_out(b):
        pltpu.make_async_copy(o0.at[b], out0_hbm.at[pl.ds(0, K * 16)],
                              semo[b]).wait()
        pltpu.make_async_copy(o1.at[b], out1_hbm.at[pl.ds(0, K * 16)],
                              semo[b]).wait()

    def compute(k, b):
        psb, pdb, o0b, o1b = ps.at[b], pd.at[b], o0.at[b], o1.at[b]

        @plsc.parallel_loop(0, K, 1)
        def edge(e):
            acc0 = jnp.zeros((16,), F32)
            acc1 = jnp.zeros((16,), F32)
            for j in range(H // 16):
                a = psb[e, pl.ds(j * 16, 16)] + pdb[e, pl.ds(j * 16, 16)]
                a = jnp.maximum(a, 0.0)
                acc0 = acc0 + a * w0[j]
                acc1 = acc1 + a * w1[j]
            o0b[pl.ds(e * 16, 16)] = acc0
            o1b[pl.ds(e * 16, 16)] = acc1

    issue(0, 0)
    issue(1, 1)

    def body(k2, carry):
        for b in range(2):
            k = k2 * 2 + b
            drain_g(b)

            @pl.when(k >= 2)
            def _():
                drain_out(b)

            compute(k, b)
            issue_out(k, b)

            @pl.when(k + 2 < CH2)
            def _():
                issue(k + 2, b)
        return carry

    lax.fori_loop(0, (CH2 - 1) // 2, body, 0)
    drain_g(0)
    drain_out(0)
    compute(CH2 - 1, 0)
    issue_out(CH2 - 1, 0)
    drain_out(0)
    drain_out(1)


def _t1_body(x_ref, w1_ref, hist_ref, t_ref, dinv_ref):
    h = hist_ref[...]
    deg = h[0] + h[1] + 1.0
    dinv = lax.rsqrt(deg)
    m = jnp.dot(x_ref[...], w1_ref[...], preferred_element_type=F32)
    m = m * dinv[:N]
    t_ref[0:N, :] = m[:, :HF]
    t_ref[NP:NP + N, :] = m[:, HF:]
    dinv_ref[...] = dinv


def _t23_body(acc_ref, t_ref, dinv_ref, b_ref, w_ref, b2_ref, out_ref,
              *, last):
    lo = acc_ref[0:N, :] + t_ref[0:N, :]
    hi = acc_ref[NP:NP + N, :] + t_ref[NP:NP + N, :]
    full = jnp.concatenate([lo, hi], axis=1)
    dinv = dinv_ref[0:N, :]
    h = jnp.maximum(full * dinv + b_ref[...], 0.0)
    m = jnp.dot(h, w_ref[...], preferred_element_type=F32)
    if last:
        out_ref[...] = m + 0.5 * b2_ref[...]
    else:
        m = m * dinv
        out_ref[0:N, :] = m[:, :HF]
        out_ref[NP:NP + N, :] = m[:, HF:]


def _t4_body(l0_ref, l1_ref, bm2_ref, out_ref):
    s0 = jnp.sum(l0_ref[...], axis=1, keepdims=True) + bm2_ref[0, 0]
    s1 = jnp.sum(l1_ref[...], axis=1, keepdims=True) + bm2_ref[0, 1]
    m = jnp.maximum(s0, s1)
    lse = m + jnp.log(jnp.exp(s0 - m) + jnp.exp(s1 - m))
    out_ref[...] = jnp.concatenate([s0 - lse, s1 - lse], axis=1)


def kernel(x, edge_index, W1, b1, W2, b2, Wm1, bm1, Wm2, bm2):
    src = edge_index[0].astype(jnp.int32)
    dst = edge_index[1].astype(jnp.int32)
    src4 = src.reshape(NW, EW2)
    dst4 = dst.reshape(NW, EW2)
    z1 = jnp.zeros((RPW,), F32)
    zrows = jnp.zeros((RPW, HF), F32)

    hist = _hist_k(dst, z1).reshape(NC, NP, 1)

    t1, dinv = pl.pallas_call(
        _t1_body,
        out_shape=[jax.ShapeDtypeStruct((NC * NP, HF), F32),
                   jax.ShapeDtypeStruct((NP, 1), F32)],
    )(x, W1, hist)

    acc1 = _scatter_k(t1, src, dst, zrows)

    t2 = pl.pallas_call(
        functools.partial(_t23_body, last=False),
        out_shape=jax.ShapeDtypeStruct((NC * NP, HF), F32),
    )(acc1, t1, dinv, b1.reshape(1, H), W2, b1.reshape(1, H))

    acc2 = _scatter_k(t2, src, dst, zrows)

    p = pl.pallas_call(
        functools.partial(_t23_body, last=True),
        out_shape=jax.ShapeDtypeStruct((N, H), F32),
    )(acc2, t2, dinv, b2.reshape(1, H), Wm1, bm1.reshape(1, H))

    l0, l1 = _edge_mlp_k(p, src4, dst4, Wm2.T)
    l0 = l0.reshape(E, 16)
    l1 = l1.reshape(E, 16)

    eb = 4000
    out = pl.pallas_call(
        _t4_body,
        grid=(E // eb,),
        in_specs=[pl.BlockSpec((eb, 16), lambda i: (i, 0)),
                  pl.BlockSpec((eb, 16), lambda i: (i, 0)),
                  pl.BlockSpec((1, 2), lambda i: (0, 0))],
        out_specs=pl.BlockSpec((eb, 2), lambda i: (i, 0)),
        out_shape=jax.ShapeDtypeStruct((E, 2), F32),
    )(l0, l1, bm2.reshape(1, 2))
    return out
